# baseline (speedup 1.0000x reference)
import functools

import jax
import jax.numpy as jnp
from jax import lax
from jax.experimental import pallas as pl
from jax.experimental.pallas import tpu as pltpu
from jax.experimental.pallas import tpu_sc as plsc

T_MAX = 4096
D_MODEL = 64
V_ROWS = 100000

NUM_CORES = 2
NUM_SUBCORES = 16
NW = NUM_CORES * NUM_SUBCORES
CHUNK = 80
NCHUNKS = V_ROWS // CHUNK
KMAX = -(-NCHUNKS // NW)
PE_W = T_MAX // NW


def _build_kernel():
    mesh = plsc.VectorSubcoreMesh(
        core_axis_name="c", subcore_axis_name="s",
        num_cores=NUM_CORES, num_subcores=NUM_SUBCORES)

    @functools.partial(
        pl.kernel,
        mesh=mesh,
        compiler_params=pltpu.CompilerParams(use_tc_tiling_on_sc=False),
        out_type=(
            jax.ShapeDtypeStruct((T_MAX, D_MODEL), jnp.float32),
            jax.ShapeDtypeStruct((V_ROWS, D_MODEL), jnp.float32),
        ),
        scratch_types=[
            pltpu.VMEM((KMAX, CHUNK), jnp.int32),
            pltpu.VMEM((CHUNK, D_MODEL), jnp.float32),
            pltpu.VMEM((PE_W, D_MODEL), jnp.float32),
            pltpu.SemaphoreType.DMA,
        ],
    )
    def enc(pe_hbm, tab_hbm, idx_hbm, pe_out, spat_out, idx_v, rows_v, pe_v, gsem):
        wid = lax.axis_index("s") * NUM_CORES + lax.axis_index("c")

        pltpu.sync_copy(idx_hbm.at[wid], idx_v)

        pe_lo = wid * PE_W
        pltpu.sync_copy(pe_hbm.at[pl.ds(pe_lo, PE_W), :], pe_v)
        pltpu.sync_copy(pe_v, pe_out.at[pl.ds(pe_lo, PE_W), :])

        @pl.loop(0, KMAX)
        def _chunk(k):
            c = wid + k * NW
            @pl.when(c < NCHUNKS)
            def _():
                pltpu.async_copy(tab_hbm.at[idx_v.at[k]], rows_v, gsem).wait()
                pltpu.sync_copy(
                    rows_v, spat_out.at[pl.ds(c * CHUNK, CHUNK), :])

    return enc


_ENC = None


def kernel(pos_encoding, spatial_table, T, V):
    global _ENC
    if _ENC is None:
        _ENC = _build_kernel()
    offset = jnp.asarray(V, jnp.int32) - jnp.int32(V_ROWS)
    node_idx = jnp.arange(V_ROWS, dtype=jnp.int32) + offset
    chunks = node_idx.reshape(NCHUNKS, CHUNK)
    pad = NW * KMAX - NCHUNKS
    chunks = jnp.concatenate([chunks, chunks[:pad]], axis=0)
    idx_wm = chunks.reshape(KMAX, NW, CHUNK).transpose(1, 0, 2)
    temporal, spatial = _ENC(pos_encoding, spatial_table, idx_wm)
    return temporal, spatial
